# baseline (speedup 1.0000x reference)
import functools

import jax
import jax.numpy as jnp
from jax import lax
from jax.experimental import pallas as pl
from jax.experimental.pallas import tpu as pltpu
from jax.experimental.pallas import tpu_sc as plsc

_N = 50176
_BLK = 1792
_GRID = _N // _BLK

_D = 1664

_NW = 32
_PW = _N // _NW
_C = 16
_ROUNDS = _PW // _C


def _lk(x):
    return jnp.maximum(x, 0.01 * x)


def _argmax2d(x, width):
    m = jnp.max(x, axis=1, keepdims=True)
    io = lax.broadcasted_iota(jnp.int32, x.shape, 1)
    return jnp.min(jnp.where(x == m, io, width), axis=1, keepdims=True)


def _dotT(xt, w):
    return lax.dot_general(xt, w, (((0,), (0,)), ((), ())),
                           preferred_element_type=jnp.float32)


def _tc_body(xf_ref,
             bb1_w, bb1_b, bb2_w, bb2_b, bb3_w, bb3_b,
             msk1_w, msk1_b, msk2_w, msk2_b, msk3_wT, msk3_b,
             c1a_w, c1a_b, c1b_w, c1b_b, c1c_w, c1c_b,
             c2aT, c2a_b, c2bT, c2b_b,
             r1_w, r1_b, r2T,
             mask_o, l_o, i12_o, r2_o):
    f32 = jnp.float32
    xt = xf_ref[...]
    x = _lk(_dotT(xt, bb1_w[...]) + bb1_b[...])
    x = _lk(jnp.dot(x, bb2_w[...], preferred_element_type=f32) + bb2_b[...])
    x_l = _lk(jnp.dot(x, bb3_w[...], preferred_element_type=f32) + bb3_b[...])
    m = _lk(_dotT(xt, msk1_w[...]) + msk1_b[...])
    m = _lk(jnp.dot(m, msk2_w[...], preferred_element_type=f32) + msk2_b[...])
    mask = _lk(jnp.sum(m * msk3_wT[...], axis=1, keepdims=True) + msk3_b[...])
    mask_o[...] = mask
    l = _lk(jnp.dot(x_l, c1a_w[...], preferred_element_type=f32) + c1a_b[...])
    l = _lk(jnp.dot(l, c1b_w[...], preferred_element_type=f32) + c1b_b[...])
    logits1 = jnp.dot(l, c1c_w[...], preferred_element_type=f32) + c1c_b[...]
    inds1 = _argmax2d(logits1, 16)
    l_o[...] = l
    allo_a = jnp.dot(l, c2aT[...], preferred_element_type=f32)
    h2 = jnp.zeros(allo_a.shape[:1] + (32,), f32)
    for k in range(16):
        mk = inds1 == k
        h2 = jnp.where(mk, allo_a[:, k * 32:(k + 1) * 32] + c2a_b[k:k + 1, :], h2)
    h2 = _lk(h2)
    allo_b = jnp.dot(h2, c2bT[...], preferred_element_type=f32)
    logits2 = jnp.zeros(allo_b.shape[:1] + (32,), f32)
    for k in range(16):
        mk = inds1 == k
        logits2 = jnp.where(mk, allo_b[:, k * 32:(k + 1) * 32] + c2b_b[k:k + 1, :],
                            logits2)
    i2 = _argmax2d(logits2, 32) - 8
    i12 = jnp.clip(inds1 * 16 + i2, 0, 255)
    i12_o[...] = i12
    xr = _lk(_dotT(xt, r1_w[...]) + r1_b[...])
    r2_o[...] = jnp.dot(xr, r2T[...], preferred_element_type=f32)


def _tc_stage(xf, bb1_w, bb1_b, bb2_w, bb2_b, bb3_w, bb3_b,
              msk1_w, msk1_b, msk2_w, msk2_b, msk3_wT, msk3_b,
              c1a_w, c1a_b, c1b_w, c1b_b, c1c_w, c1c_b,
              c2aT, c2a_b, c2bT, c2b_b, r1_w, r1_b, r2T):
    full = lambda a: pl.BlockSpec(a.shape, lambda i: (0,) * a.ndim)
    ins = (bb1_w, bb1_b, bb2_w, bb2_b, bb3_w, bb3_b,
           msk1_w, msk1_b, msk2_w, msk2_b, msk3_wT, msk3_b,
           c1a_w, c1a_b, c1b_w, c1b_b, c1c_w, c1c_b,
           c2aT, c2a_b, c2bT, c2b_b, r1_w, r1_b, r2T)
    return pl.pallas_call(
        _tc_body,
        grid=(_GRID,),
        in_specs=[pl.BlockSpec((128, _BLK), lambda i: (0, i))] +
                 [full(a) for a in ins],
        out_specs=[
            pl.BlockSpec((_BLK, 1), lambda i: (i, 0)),
            pl.BlockSpec((_BLK, 32), lambda i: (i, 0)),
            pl.BlockSpec((_BLK, 1), lambda i: (i, 0)),
            pl.BlockSpec((_BLK, 256), lambda i: (i, 0)),
        ],
        out_shape=[
            jax.ShapeDtypeStruct((_N, 1), jnp.float32),
            jax.ShapeDtypeStruct((_N, 32), jnp.float32),
            jax.ShapeDtypeStruct((_N, 1), jnp.int32),
            jax.ShapeDtypeStruct((_N, 256), jnp.float32),
        ],
        compiler_params=pltpu.CompilerParams(
            fuse_transposed_lhs_in_matmul=True),
    )(xf, *ins)


def _sc_body(l_hbm, i12_hbm, r2_hbm, m_hbm, r2b_hbm, out_hbm,
             idx_all, mrows2, l2, r2_1, r2b_v, out_all, hb, lgb,
             sem_g, sem_i, sem_r):
    wid = lax.axis_index("s") * 2 + lax.axis_index("c")
    base_pix = wid * _PW
    pltpu.sync_copy(r2b_hbm, r2b_v)
    pltpu.sync_copy(i12_hbm.at[pl.ds(base_pix, _PW)], idx_all)
    io16 = lax.iota(jnp.int32, 16)

    def _splat(v):
        return jnp.zeros((16,), jnp.int32) + v

    def _up(w):
        return plsc.unpack(plsc.bitcast(w, jnp.bfloat16),
                           format=plsc.PackFormat.INTERLEAVED)

    def issue_g(r):
        pltpu.async_copy(m_hbm.at[idx_all.at[pl.ds(r * _C, _C)]],
                         mrows2.at[r % 2], sem_g)

    def issue_l(r):
        pltpu.async_copy(l_hbm.at[pl.ds(base_pix + r * _C, _C)],
                         l2.at[r % 2], sem_i)

    def issue_r2(r):
        pltpu.async_copy(r2_hbm.at[pl.ds(base_pix + r * _C, _C)], r2_1, sem_r)

    issue_g(0)
    issue_g(1)
    issue_l(0)
    issue_l(1)
    issue_r2(0)

    def round_body(r, carry):
        pb = r % 2
        start = base_pix + r * _C
        pltpu.make_async_copy(m_hbm.at[idx_all.at[pl.ds(r * _C, _C)]],
                              mrows2.at[pb], sem_g).wait()
        pltpu.make_async_copy(l_hbm.at[pl.ds(start, _C)], l2.at[pb],
                              sem_i).wait()
        e_vec = idx_all[pl.ds(r * _C, _C)]
        oz = jnp.clip(jnp.max(e_vec), 0, 0)
        pbs = _splat(pb)

        def quad(q, c):
            for j in range(4):
                p = q * 4 + j
                ps = _splat(p)
                a0, a1 = _up(mrows2[pb, p, pl.ds(512, 16)])
                a2 = jnp.zeros((16,), jnp.float32)
                a3 = jnp.zeros((16,), jnp.float32)
                for i in range(32):
                    xi = plsc.load_gather(l2, [pbs, ps, _splat(i)])
                    w0, w1 = _up(mrows2[pb, p, pl.ds(i * 16, 16)])
                    if i % 2 == 0:
                        a0 = a0 + xi * w0
                        a1 = a1 + xi * w1
                    else:
                        a2 = a2 + xi * w0
                        a3 = a3 + xi * w1
                hb[p, pl.ds(0, 16)] = _lk(a0 + a2)
                hb[p, pl.ds(16, 16)] = _lk(a1 + a3)
                b0, b1 = _up(mrows2[pb, p, pl.ds(1040, 16)])
                b2 = jnp.zeros((16,), jnp.float32)
                b3 = jnp.zeros((16,), jnp.float32)
                for i in range(32):
                    hi = plsc.load_gather(hb, [ps, _splat(i)])
                    w0, w1 = _up(mrows2[pb, p, pl.ds(528 + i * 16, 16)])
                    if i % 2 == 0:
                        b0 = b0 + hi * w0
                        b1 = b1 + hi * w1
                    else:
                        b2 = b2 + hi * w0
                        b3 = b3 + hi * w1
                lgb[p, pl.ds(0, 16)] = b0 + b2
                lgb[p, pl.ds(16, 16)] = b1 + b3
            return c

        lax.fori_loop(0, 4 + oz, quad, 0)

        mx = plsc.load_gather(lgb, [io16, _splat(0)])
        for o in range(1, 32):
            mx = jnp.maximum(mx, plsc.load_gather(lgb, [io16, _splat(o)]))
        am = _splat(64)
        for o in range(32):
            lo = plsc.load_gather(lgb, [io16, _splat(o)])
            am = jnp.minimum(am, jnp.where(lo == mx, o, 64))
        inds_v = jnp.clip(e_vec * 16 + (am - 8), 0, 4095)
        off = inds_v - jnp.clip(e_vec * 16 - 8, 0, 4064)
        sup = inds_v // 512
        pltpu.make_async_copy(r2_hbm.at[pl.ds(start, _C)], r2_1,
                              sem_r).wait()
        sup32 = sup * 32
        roff = 1056 + off * 16
        racc = [jnp.zeros((16,), jnp.float32) for _ in range(4)]
        for k in range(16):
            xlo = _lk(plsc.load_gather(r2_1, [io16, sup32 + k]) +
                      plsc.load_gather(r2b_v, [sup, _splat(k)]))
            xhi = _lk(plsc.load_gather(r2_1, [io16, sup32 + k + 16]) +
                      plsc.load_gather(r2b_v, [sup, _splat(k + 16)]))
            wlo, whi = _up(plsc.load_gather(mrows2, [pbs, io16, roff + k]))
            racc[2 * (k % 2)] = racc[2 * (k % 2)] + xlo * wlo
            racc[2 * (k % 2) + 1] = racc[2 * (k % 2) + 1] + xhi * whi
        rb = plsc.load_gather(mrows2, [pbs, io16, 1568 + off])
        rr = (racc[0] + racc[1]) + (racc[2] + racc[3]) + rb
        out_all[pl.ds(r * _C, _C)] = (
            (inds_v.astype(jnp.float32) + rr) * (1.0 / 4096.0))

        @pl.when(r + 1 < _ROUNDS)
        def _():
            issue_r2(r + 1)

        @pl.when(r + 2 < _ROUNDS)
        def _():
            issue_g(r + 2)
            issue_l(r + 2)

        return carry

    lax.fori_loop(0, _ROUNDS, round_body, 0)
    pltpu.sync_copy(out_all, out_hbm.at[pl.ds(base_pix, _PW)])


def _sc_stage(l, i12, r2all, mtab, r2_b):
    mesh = plsc.VectorSubcoreMesh(core_axis_name="c", subcore_axis_name="s")
    f = functools.partial(
        pl.kernel,
        out_type=jax.ShapeDtypeStruct((_N,), jnp.float32),
        mesh=mesh,
        scratch_types=[
            pltpu.VMEM((_PW,), jnp.int32),
            pltpu.VMEM((2, _C, _D), jnp.float32),
            pltpu.VMEM((2, _C, 32), jnp.float32),
            pltpu.VMEM((_C, 256), jnp.float32),
            pltpu.VMEM((8, 32), jnp.float32),
            pltpu.VMEM((_PW,), jnp.float32),
            pltpu.VMEM((16, 32), jnp.float32),
            pltpu.VMEM((16, 32), jnp.float32),
            pltpu.SemaphoreType.DMA,
            pltpu.SemaphoreType.DMA,
            pltpu.SemaphoreType.DMA,
        ],
        compiler_params=pltpu.CompilerParams(needs_layout_passes=False),
    )(_sc_body)
    return f(l, i12, r2all, mtab, r2_b)


def kernel(x_in, bb1_w, bb1_b, bb2_w, bb2_b, bb3_w, bb3_b,
           msk1_w, msk1_b, msk2_w, msk2_b, msk3_w, msk3_b,
           c1a_w, c1a_b, c1b_w, c1b_b, c1c_w, c1c_b,
           c2a_w, c2a_b, c2b_w, c2b_b, c3a_w, c3a_b, c3b_w, c3b_b,
           r1_w, r1_b, r2_w, r2_b, r3_w, r3_b):
    b, ch, hh, ww = x_in.shape
    xf = x_in.reshape(ch, _N)
    c2aT = jnp.transpose(c2a_w, (1, 0, 2)).reshape(32, 512)
    c2bT = jnp.transpose(c2b_w, (1, 0, 2)).reshape(32, 512)
    r2T = jnp.transpose(r2_w, (1, 0, 2)).reshape(128, 256)
    msk3_wT = msk3_w.reshape(1, 16)
    def _pack_halves(x):
        lo = x[..., :16].astype(jnp.bfloat16)
        hi = x[..., 16:].astype(jnp.bfloat16)
        return jax.lax.bitcast_convert_type(
            jnp.stack([lo, hi], axis=-1), jnp.float32)

    j = jnp.arange(256, dtype=jnp.int32)
    rbase = jnp.clip(j * 16 - 8, 0, 4096 - 32)
    rows = rbase[:, None] + jnp.arange(32, dtype=jnp.int32)[None, :]
    r3flat = r3_w[:, :, 0]
    rblk = r3flat[rows]
    rbblk = r3_b[:, 0][rows]
    mtab = jnp.concatenate(
        [_pack_halves(c3a_w).reshape(256, 512), _pack_halves(c3a_b),
         _pack_halves(c3b_w).reshape(256, 512), _pack_halves(c3b_b),
         _pack_halves(rblk).reshape(256, 512), rbblk,
         jnp.zeros((256, 64), jnp.float32)], axis=1)

    mask_f, l, i12, r2all = _tc_stage(
        xf, bb1_w, bb1_b.reshape(1, 128), bb2_w, bb2_b.reshape(1, 128),
        bb3_w, bb3_b.reshape(1, 128),
        msk1_w, msk1_b.reshape(1, 32), msk2_w, msk2_b.reshape(1, 16),
        msk3_wT, msk3_b.reshape(1, 1),
        c1a_w, c1a_b.reshape(1, 32), c1b_w, c1b_b.reshape(1, 32),
        c1c_w, c1c_b.reshape(1, 16),
        c2aT, c2a_b, c2bT, c2b_b,
        r1_w, r1_b.reshape(1, 128), r2T)

    out_flat = _sc_stage(l, i12.reshape(_N), r2all, mtab, r2_b)
    out = out_flat.reshape(b, 1, hh, ww)
    mask = mask_f.reshape(b, 1, hh, ww)
    return out, mask


# speedup vs baseline: 1.3063x; 1.3063x over previous
import functools

import jax
import jax.numpy as jnp
from jax import lax
from jax.experimental import pallas as pl
from jax.experimental.pallas import tpu as pltpu
from jax.experimental.pallas import tpu_sc as plsc

_N = 50176
_BLK = 1792
_GRID = _N // _BLK

_D = 1664

_NW = 32
_PW = _N // _NW
_C = 16
_ROUNDS = _PW // _C


def _lk(x):
    return jnp.maximum(x, 0.01 * x)


def _argmax2d(x, width):
    m = jnp.max(x, axis=1, keepdims=True)
    io = lax.broadcasted_iota(jnp.int32, x.shape, 1)
    return jnp.min(jnp.where(x == m, io, width), axis=1, keepdims=True)


def _tc_body(xf_ref,
             bb1_w, bb1_b, bb2_w, bb2_b, bb3_w, bb3_b,
             msk1_w, msk1_b, msk2_w, msk2_b, msk3_wT, msk3_b,
             c1a_w, c1a_b, c1b_w, c1b_b, c1c_w, c1c_b,
             c2aT, c2a_b, c2bT, c2b_b,
             r1_w, r1_b, r2T,
             mask_o, l_o, i12_o, r2_o):
    f32 = jnp.float32
    xf = xf_ref[...]
    x = _lk(jnp.dot(xf, bb1_w[...], preferred_element_type=f32) + bb1_b[...])
    x = _lk(jnp.dot(x, bb2_w[...], preferred_element_type=f32) + bb2_b[...])
    x_l = _lk(jnp.dot(x, bb3_w[...], preferred_element_type=f32) + bb3_b[...])
    m = _lk(jnp.dot(xf, msk1_w[...], preferred_element_type=f32) + msk1_b[...])
    m = _lk(jnp.dot(m, msk2_w[...], preferred_element_type=f32) + msk2_b[...])
    mask = _lk(jnp.sum(m * msk3_wT[...], axis=1, keepdims=True) + msk3_b[...])
    mask_o[...] = mask
    l = _lk(jnp.dot(x_l, c1a_w[...], preferred_element_type=f32) + c1a_b[...])
    l = _lk(jnp.dot(l, c1b_w[...], preferred_element_type=f32) + c1b_b[...])
    logits1 = jnp.dot(l, c1c_w[...], preferred_element_type=f32) + c1c_b[...]
    inds1 = _argmax2d(logits1, 16)
    l_o[...] = l
    allo_a = jnp.dot(l, c2aT[...], preferred_element_type=f32)
    h2 = jnp.zeros(allo_a.shape[:1] + (32,), f32)
    for k in range(16):
        mk = inds1 == k
        h2 = jnp.where(mk, allo_a[:, k * 32:(k + 1) * 32] + c2a_b[k:k + 1, :], h2)
    h2 = _lk(h2)
    allo_b = jnp.dot(h2, c2bT[...], preferred_element_type=f32)
    logits2 = jnp.zeros(allo_b.shape[:1] + (32,), f32)
    for k in range(16):
        mk = inds1 == k
        logits2 = jnp.where(mk, allo_b[:, k * 32:(k + 1) * 32] + c2b_b[k:k + 1, :],
                            logits2)
    i2 = _argmax2d(logits2, 32) - 8
    i12 = jnp.clip(inds1 * 16 + i2, 0, 255)
    i12_o[...] = i12
    xr = _lk(jnp.dot(xf, r1_w[...], preferred_element_type=f32) + r1_b[...])
    r2_o[...] = jnp.dot(xr, r2T[...], preferred_element_type=f32)


def _tc_stage(xf, bb1_w, bb1_b, bb2_w, bb2_b, bb3_w, bb3_b,
              msk1_w, msk1_b, msk2_w, msk2_b, msk3_wT, msk3_b,
              c1a_w, c1a_b, c1b_w, c1b_b, c1c_w, c1c_b,
              c2aT, c2a_b, c2bT, c2b_b, r1_w, r1_b, r2T):
    full = lambda a: pl.BlockSpec(a.shape, lambda i: (0,) * a.ndim)
    ins = (bb1_w, bb1_b, bb2_w, bb2_b, bb3_w, bb3_b,
           msk1_w, msk1_b, msk2_w, msk2_b, msk3_wT, msk3_b,
           c1a_w, c1a_b, c1b_w, c1b_b, c1c_w, c1c_b,
           c2aT, c2a_b, c2bT, c2b_b, r1_w, r1_b, r2T)
    n = xf.shape[0]
    return pl.pallas_call(
        _tc_body,
        grid=(n // _BLK,),
        in_specs=[pl.BlockSpec((_BLK, 128), lambda i: (i, 0))] +
                 [full(a) for a in ins],
        out_specs=[
            pl.BlockSpec((_BLK, 1), lambda i: (i, 0)),
            pl.BlockSpec((_BLK, 32), lambda i: (i, 0)),
            pl.BlockSpec((_BLK, 1), lambda i: (i, 0)),
            pl.BlockSpec((_BLK, 256), lambda i: (i, 0)),
        ],
        out_shape=[
            jax.ShapeDtypeStruct((n, 1), jnp.float32),
            jax.ShapeDtypeStruct((n, 32), jnp.float32),
            jax.ShapeDtypeStruct((n, 1), jnp.int32),
            jax.ShapeDtypeStruct((n, 256), jnp.float32),
        ],
    )(xf, *ins)


def _sc_body(l_hbm, i12_hbm, r2_hbm, m_hbm, r2b_hbm, out_hbm,
             idx_all, mrows2, l2, r2_1, r2b_v, out_all, hb, lgb,
             sem_g, sem_i, sem_r, *, pw, rounds):
    wid = lax.axis_index("s") * 2 + lax.axis_index("c")
    base_pix = wid * pw
    pltpu.sync_copy(r2b_hbm, r2b_v)
    pltpu.sync_copy(i12_hbm.at[pl.ds(base_pix, pw)], idx_all)
    io16 = lax.iota(jnp.int32, 16)

    def _splat(v):
        return jnp.zeros((16,), jnp.int32) + v

    def _up(w):
        return plsc.unpack(plsc.bitcast(w, jnp.bfloat16),
                           format=plsc.PackFormat.INTERLEAVED)

    def issue_g(r):
        pltpu.async_copy(m_hbm.at[idx_all.at[pl.ds(r * _C, _C)]],
                         mrows2.at[r % 2], sem_g)

    def issue_l(r):
        pltpu.async_copy(l_hbm.at[pl.ds(base_pix + r * _C, _C)],
                         l2.at[r % 2], sem_i)

    def issue_r2(r):
        pltpu.async_copy(r2_hbm.at[pl.ds(base_pix + r * _C, _C)], r2_1, sem_r)

    issue_g(0)
    issue_g(1)
    issue_l(0)
    issue_l(1)
    issue_r2(0)

    def round_body(r, carry):
        pb = r % 2
        start = base_pix + r * _C
        pltpu.make_async_copy(m_hbm.at[idx_all.at[pl.ds(r * _C, _C)]],
                              mrows2.at[pb], sem_g).wait()
        pltpu.make_async_copy(l_hbm.at[pl.ds(start, _C)], l2.at[pb],
                              sem_i).wait()
        e_vec = idx_all[pl.ds(r * _C, _C)]
        oz = jnp.clip(jnp.max(e_vec), 0, 0)
        pbs = _splat(pb)

        def quad(q, c):
            for j in range(4):
                p = q * 4 + j
                ps = _splat(p)
                a0, a1 = _up(mrows2[pb, p, pl.ds(512, 16)])
                a2 = jnp.zeros((16,), jnp.float32)
                a3 = jnp.zeros((16,), jnp.float32)
                for i in range(32):
                    xi = plsc.load_gather(l2, [pbs, ps, _splat(i)])
                    w0, w1 = _up(mrows2[pb, p, pl.ds(i * 16, 16)])
                    if i % 2 == 0:
                        a0 = a0 + xi * w0
                        a1 = a1 + xi * w1
                    else:
                        a2 = a2 + xi * w0
                        a3 = a3 + xi * w1
                hb[p, pl.ds(0, 16)] = _lk(a0 + a2)
                hb[p, pl.ds(16, 16)] = _lk(a1 + a3)
                b0, b1 = _up(mrows2[pb, p, pl.ds(1040, 16)])
                b2 = jnp.zeros((16,), jnp.float32)
                b3 = jnp.zeros((16,), jnp.float32)
                for i in range(32):
                    hi = plsc.load_gather(hb, [ps, _splat(i)])
                    w0, w1 = _up(mrows2[pb, p, pl.ds(528 + i * 16, 16)])
                    if i % 2 == 0:
                        b0 = b0 + hi * w0
                        b1 = b1 + hi * w1
                    else:
                        b2 = b2 + hi * w0
                        b3 = b3 + hi * w1
                lgb[p, pl.ds(0, 16)] = b0 + b2
                lgb[p, pl.ds(16, 16)] = b1 + b3
            return c

        lax.fori_loop(0, 4 + oz, quad, 0)

        mx = plsc.load_gather(lgb, [io16, _splat(0)])
        for o in range(1, 32):
            mx = jnp.maximum(mx, plsc.load_gather(lgb, [io16, _splat(o)]))
        am = _splat(64)
        for o in range(32):
            lo = plsc.load_gather(lgb, [io16, _splat(o)])
            am = jnp.minimum(am, jnp.where(lo == mx, o, 64))
        inds_v = jnp.clip(e_vec * 16 + (am - 8), 0, 4095)
        off = inds_v - jnp.clip(e_vec * 16 - 8, 0, 4064)
        sup = inds_v // 512
        pltpu.make_async_copy(r2_hbm.at[pl.ds(start, _C)], r2_1,
                              sem_r).wait()
        sup32 = sup * 32
        roff = 1056 + off * 16
        racc = [jnp.zeros((16,), jnp.float32) for _ in range(4)]
        for k in range(16):
            xlo = _lk(plsc.load_gather(r2_1, [io16, sup32 + k]) +
                      plsc.load_gather(r2b_v, [sup, _splat(k)]))
            xhi = _lk(plsc.load_gather(r2_1, [io16, sup32 + k + 16]) +
                      plsc.load_gather(r2b_v, [sup, _splat(k + 16)]))
            wlo, whi = _up(plsc.load_gather(mrows2, [pbs, io16, roff + k]))
            racc[2 * (k % 2)] = racc[2 * (k % 2)] + xlo * wlo
            racc[2 * (k % 2) + 1] = racc[2 * (k % 2) + 1] + xhi * whi
        rb = plsc.load_gather(mrows2, [pbs, io16, 1568 + off])
        rr = (racc[0] + racc[1]) + (racc[2] + racc[3]) + rb
        out_all[pl.ds(r * _C, _C)] = (
            (inds_v.astype(jnp.float32) + rr) * (1.0 / 4096.0))

        @pl.when(r + 1 < rounds)
        def _():
            issue_r2(r + 1)

        @pl.when(r + 2 < rounds)
        def _():
            issue_g(r + 2)
            issue_l(r + 2)

        return carry

    lax.fori_loop(0, rounds, round_body, 0)
    pltpu.sync_copy(out_all, out_hbm.at[pl.ds(base_pix, pw)])


def _sc_stage(l, i12, r2all, mtab, r2_b):
    n = l.shape[0]
    pw = n // _NW
    rounds = pw // _C
    mesh = plsc.VectorSubcoreMesh(core_axis_name="c", subcore_axis_name="s")
    body = functools.partial(_sc_body, pw=pw, rounds=rounds)
    f = functools.partial(
        pl.kernel,
        out_type=jax.ShapeDtypeStruct((n,), jnp.float32),
        mesh=mesh,
        scratch_types=[
            pltpu.VMEM((pw,), jnp.int32),
            pltpu.VMEM((2, _C, _D), jnp.float32),
            pltpu.VMEM((2, _C, 32), jnp.float32),
            pltpu.VMEM((_C, 256), jnp.float32),
            pltpu.VMEM((8, 32), jnp.float32),
            pltpu.VMEM((pw,), jnp.float32),
            pltpu.VMEM((16, 32), jnp.float32),
            pltpu.VMEM((16, 32), jnp.float32),
            pltpu.SemaphoreType.DMA,
            pltpu.SemaphoreType.DMA,
            pltpu.SemaphoreType.DMA,
        ],
        compiler_params=pltpu.CompilerParams(needs_layout_passes=False),
    )(body)
    return f(l, i12, r2all, mtab, r2_b)


def kernel(x_in, bb1_w, bb1_b, bb2_w, bb2_b, bb3_w, bb3_b,
           msk1_w, msk1_b, msk2_w, msk2_b, msk3_w, msk3_b,
           c1a_w, c1a_b, c1b_w, c1b_b, c1c_w, c1c_b,
           c2a_w, c2a_b, c2b_w, c2b_b, c3a_w, c3a_b, c3b_w, c3b_b,
           r1_w, r1_b, r2_w, r2_b, r3_w, r3_b):
    b, ch, hh, ww = x_in.shape
    xf = jnp.transpose(x_in, (0, 2, 3, 1)).reshape(_N, ch)
    c2aT = jnp.transpose(c2a_w, (1, 0, 2)).reshape(32, 512)
    c2bT = jnp.transpose(c2b_w, (1, 0, 2)).reshape(32, 512)
    r2T = jnp.transpose(r2_w, (1, 0, 2)).reshape(128, 256)
    msk3_wT = msk3_w.reshape(1, 16)
    def _pack_halves(x):
        lo = x[..., :16].astype(jnp.bfloat16)
        hi = x[..., 16:].astype(jnp.bfloat16)
        return jax.lax.bitcast_convert_type(
            jnp.stack([lo, hi], axis=-1), jnp.float32)

    j = jnp.arange(256, dtype=jnp.int32)
    rbase = jnp.clip(j * 16 - 8, 0, 4096 - 32)
    rows = rbase[:, None] + jnp.arange(32, dtype=jnp.int32)[None, :]
    r3flat = r3_w[:, :, 0]
    rblk = r3flat[rows]
    rbblk = r3_b[:, 0][rows]
    mtab = jnp.concatenate(
        [_pack_halves(c3a_w).reshape(256, 512), _pack_halves(c3a_b),
         _pack_halves(c3b_w).reshape(256, 512), _pack_halves(c3b_b),
         _pack_halves(rblk).reshape(256, 512), rbblk,
         jnp.zeros((256, 64), jnp.float32)], axis=1)

    nh = _N // 2
    outs = []
    masks = []
    for h in range(2):
        xf_h = xf[h * nh:(h + 1) * nh]
        mask_f, l, i12, r2all = _tc_stage(
            xf_h, bb1_w, bb1_b.reshape(1, 128), bb2_w, bb2_b.reshape(1, 128),
            bb3_w, bb3_b.reshape(1, 128),
            msk1_w, msk1_b.reshape(1, 32), msk2_w, msk2_b.reshape(1, 16),
            msk3_wT, msk3_b.reshape(1, 1),
            c1a_w, c1a_b.reshape(1, 32), c1b_w, c1b_b.reshape(1, 32),
            c1c_w, c1c_b.reshape(1, 16),
            c2aT, c2a_b, c2bT, c2b_b,
            r1_w, r1_b.reshape(1, 128), r2T)
        outs.append(_sc_stage(l, i12.reshape(nh), r2all, mtab, r2_b))
        masks.append(mask_f)
    out = jnp.concatenate(outs).reshape(b, 1, hh, ww)
    mask = jnp.concatenate(masks).reshape(b, 1, hh, ww)
    return out, mask


# speedup vs baseline: 1.3556x; 1.0377x over previous
import functools

import jax
import jax.numpy as jnp
from jax import lax
from jax.experimental import pallas as pl
from jax.experimental.pallas import tpu as pltpu
from jax.experimental.pallas import tpu_sc as plsc

_N = 50176
_BLK = 1792
_GRID = _N // _BLK

_D = 1664

_NW = 32
_PW = _N // _NW
_C = 16
_ROUNDS = _PW // _C


def _lk(x):
    return jnp.maximum(x, 0.01 * x)


def _argmax2d(x, width):
    m = jnp.max(x, axis=1, keepdims=True)
    io = lax.broadcasted_iota(jnp.int32, x.shape, 1)
    return jnp.min(jnp.where(x == m, io, width), axis=1, keepdims=True)


def _tc_body(xf_ref,
             bb1_w, bb1_b, bb2_w, bb2_b, bb3_w, bb3_b,
             msk1_w, msk1_b, msk2_w, msk2_b, msk3_wT, msk3_b,
             c1a_w, c1a_b, c1b_w, c1b_b, c1c_w, c1c_b,
             c2aT, c2a_b, c2bT, c2b_b,
             r1_w, r1_b, r2T,
             mask_o, l_o, i12_o, r2_o):
    f32 = jnp.float32
    xf = xf_ref[...]
    x = _lk(jnp.dot(xf, bb1_w[...], preferred_element_type=f32) + bb1_b[...])
    x = _lk(jnp.dot(x, bb2_w[...], preferred_element_type=f32) + bb2_b[...])
    x_l = _lk(jnp.dot(x, bb3_w[...], preferred_element_type=f32) + bb3_b[...])
    m = _lk(jnp.dot(xf, msk1_w[...], preferred_element_type=f32) + msk1_b[...])
    m = _lk(jnp.dot(m, msk2_w[...], preferred_element_type=f32) + msk2_b[...])
    mask = _lk(jnp.sum(m * msk3_wT[...], axis=1, keepdims=True) + msk3_b[...])
    mask_o[...] = mask
    l = _lk(jnp.dot(x_l, c1a_w[...], preferred_element_type=f32) + c1a_b[...])
    l = _lk(jnp.dot(l, c1b_w[...], preferred_element_type=f32) + c1b_b[...])
    logits1 = jnp.dot(l, c1c_w[...], preferred_element_type=f32) + c1c_b[...]
    inds1 = _argmax2d(logits1, 16)
    l_o[...] = l
    allo_a = jnp.dot(l, c2aT[...], preferred_element_type=f32)
    h2 = jnp.zeros(allo_a.shape[:1] + (32,), f32)
    for k in range(16):
        mk = inds1 == k
        h2 = jnp.where(mk, allo_a[:, k * 32:(k + 1) * 32] + c2a_b[k:k + 1, :], h2)
    h2 = _lk(h2)
    allo_b = jnp.dot(h2, c2bT[...], preferred_element_type=f32)
    logits2 = jnp.zeros(allo_b.shape[:1] + (32,), f32)
    for k in range(16):
        mk = inds1 == k
        logits2 = jnp.where(mk, allo_b[:, k * 32:(k + 1) * 32] + c2b_b[k:k + 1, :],
                            logits2)
    i2 = _argmax2d(logits2, 32) - 8
    i12 = jnp.clip(inds1 * 16 + i2, 0, 255)
    i12_o[...] = i12
    xr = _lk(jnp.dot(xf, r1_w[...], preferred_element_type=f32) + r1_b[...])
    r2_o[...] = jnp.dot(xr, r2T[...], preferred_element_type=f32)


def _tc_stage(xf, bb1_w, bb1_b, bb2_w, bb2_b, bb3_w, bb3_b,
              msk1_w, msk1_b, msk2_w, msk2_b, msk3_wT, msk3_b,
              c1a_w, c1a_b, c1b_w, c1b_b, c1c_w, c1c_b,
              c2aT, c2a_b, c2bT, c2b_b, r1_w, r1_b, r2T):
    full = lambda a: pl.BlockSpec(a.shape, lambda i: (0,) * a.ndim)
    ins = (bb1_w, bb1_b, bb2_w, bb2_b, bb3_w, bb3_b,
           msk1_w, msk1_b, msk2_w, msk2_b, msk3_wT, msk3_b,
           c1a_w, c1a_b, c1b_w, c1b_b, c1c_w, c1c_b,
           c2aT, c2a_b, c2bT, c2b_b, r1_w, r1_b, r2T)
    n = xf.shape[0]
    return pl.pallas_call(
        _tc_body,
        grid=(n // _BLK,),
        in_specs=[pl.BlockSpec((_BLK, 128), lambda i: (i, 0))] +
                 [full(a) for a in ins],
        out_specs=[
            pl.BlockSpec((_BLK, 1), lambda i: (i, 0)),
            pl.BlockSpec((_BLK, 32), lambda i: (i, 0)),
            pl.BlockSpec((_BLK, 1), lambda i: (i, 0)),
            pl.BlockSpec((_BLK, 256), lambda i: (i, 0)),
        ],
        out_shape=[
            jax.ShapeDtypeStruct((n, 1), jnp.float32),
            jax.ShapeDtypeStruct((n, 32), jnp.float32),
            jax.ShapeDtypeStruct((n, 1), jnp.int32),
            jax.ShapeDtypeStruct((n, 256), jnp.float32),
        ],
    )(xf, *ins)


def _sc_body(l_hbm, i12_hbm, r2_hbm, m_hbm, r2b_hbm, out_hbm,
             idx_all, mrows2, l2, r2_1, r2b_v, out_all, hb, lgb,
             sem_g, sem_i, sem_r, *, pw, rounds):
    wid = lax.axis_index("s") * 2 + lax.axis_index("c")
    base_pix = wid * pw
    pltpu.sync_copy(r2b_hbm, r2b_v)
    pltpu.sync_copy(i12_hbm.at[pl.ds(base_pix, pw)], idx_all)
    io16 = lax.iota(jnp.int32, 16)

    def _splat(v):
        return jnp.zeros((16,), jnp.int32) + v

    def _up(w):
        return plsc.unpack(plsc.bitcast(w, jnp.bfloat16),
                           format=plsc.PackFormat.INTERLEAVED)

    def issue_g(r):
        pltpu.async_copy(m_hbm.at[idx_all.at[pl.ds(r * _C, _C)]],
                         mrows2.at[r % 2], sem_g)

    def issue_l(r):
        pltpu.async_copy(l_hbm.at[pl.ds(base_pix + r * _C, _C)],
                         l2.at[r % 2], sem_i)

    def issue_r2(r):
        pltpu.async_copy(r2_hbm.at[pl.ds(base_pix + r * _C, _C)], r2_1, sem_r)

    issue_g(0)
    issue_g(1)
    issue_l(0)
    issue_l(1)
    issue_r2(0)

    def round_body(r, carry):
        pb = r % 2
        start = base_pix + r * _C
        pltpu.make_async_copy(m_hbm.at[idx_all.at[pl.ds(r * _C, _C)]],
                              mrows2.at[pb], sem_g).wait()
        pltpu.make_async_copy(l_hbm.at[pl.ds(start, _C)], l2.at[pb],
                              sem_i).wait()
        e_vec = idx_all[pl.ds(r * _C, _C)]
        oz = jnp.clip(jnp.max(e_vec), 0, 0)
        pbs = _splat(pb)

        def quad(q, c):
            for j in range(4):
                p = q * 4 + j
                ps = _splat(p)
                a0, a1 = _up(mrows2[pb, p, pl.ds(512, 16)])
                a2 = jnp.zeros((16,), jnp.float32)
                a3 = jnp.zeros((16,), jnp.float32)
                for i in range(32):
                    xi = plsc.load_gather(l2, [pbs, ps, _splat(i)])
                    w0, w1 = _up(mrows2[pb, p, pl.ds(i * 16, 16)])
                    if i % 2 == 0:
                        a0 = a0 + xi * w0
                        a1 = a1 + xi * w1
                    else:
                        a2 = a2 + xi * w0
                        a3 = a3 + xi * w1
                hb[p, pl.ds(0, 16)] = _lk(a0 + a2)
                hb[p, pl.ds(16, 16)] = _lk(a1 + a3)
                b0, b1 = _up(mrows2[pb, p, pl.ds(1040, 16)])
                b2 = jnp.zeros((16,), jnp.float32)
                b3 = jnp.zeros((16,), jnp.float32)
                for i in range(32):
                    hi = plsc.load_gather(hb, [ps, _splat(i)])
                    w0, w1 = _up(mrows2[pb, p, pl.ds(528 + i * 16, 16)])
                    if i % 2 == 0:
                        b0 = b0 + hi * w0
                        b1 = b1 + hi * w1
                    else:
                        b2 = b2 + hi * w0
                        b3 = b3 + hi * w1
                lgb[p, pl.ds(0, 16)] = b0 + b2
                lgb[p, pl.ds(16, 16)] = b1 + b3
            return c

        lax.fori_loop(0, 4 + oz, quad, 0)

        mx = plsc.load_gather(lgb, [io16, _splat(0)])
        for o in range(1, 32):
            mx = jnp.maximum(mx, plsc.load_gather(lgb, [io16, _splat(o)]))
        am = _splat(64)
        for o in range(32):
            lo = plsc.load_gather(lgb, [io16, _splat(o)])
            am = jnp.minimum(am, jnp.where(lo == mx, o, 64))
        inds_v = jnp.clip(e_vec * 16 + (am - 8), 0, 4095)
        off = inds_v - jnp.clip(e_vec * 16 - 8, 0, 4064)
        sup = inds_v // 512
        pltpu.make_async_copy(r2_hbm.at[pl.ds(start, _C)], r2_1,
                              sem_r).wait()
        sup32 = sup * 32
        roff = 1056 + off * 16
        racc = [jnp.zeros((16,), jnp.float32) for _ in range(4)]
        for k in range(16):
            xlo = _lk(plsc.load_gather(r2_1, [io16, sup32 + k]) +
                      plsc.load_gather(r2b_v, [sup, _splat(k)]))
            xhi = _lk(plsc.load_gather(r2_1, [io16, sup32 + k + 16]) +
                      plsc.load_gather(r2b_v, [sup, _splat(k + 16)]))
            wlo, whi = _up(plsc.load_gather(mrows2, [pbs, io16, roff + k]))
            racc[2 * (k % 2)] = racc[2 * (k % 2)] + xlo * wlo
            racc[2 * (k % 2) + 1] = racc[2 * (k % 2) + 1] + xhi * whi
        rb = plsc.load_gather(mrows2, [pbs, io16, 1568 + off])
        rr = (racc[0] + racc[1]) + (racc[2] + racc[3]) + rb
        out_all[pl.ds(r * _C, _C)] = (
            (inds_v.astype(jnp.float32) + rr) * (1.0 / 4096.0))

        @pl.when(r + 1 < rounds)
        def _():
            issue_r2(r + 1)

        @pl.when(r + 2 < rounds)
        def _():
            issue_g(r + 2)
            issue_l(r + 2)

        return carry

    lax.fori_loop(0, rounds, round_body, 0)
    pltpu.sync_copy(out_all, out_hbm.at[pl.ds(base_pix, pw)])


def _sc_stage(l, i12, r2all, mtab, r2_b):
    n = l.shape[0]
    pw = n // _NW
    rounds = pw // _C
    mesh = plsc.VectorSubcoreMesh(core_axis_name="c", subcore_axis_name="s")
    body = functools.partial(_sc_body, pw=pw, rounds=rounds)
    f = functools.partial(
        pl.kernel,
        out_type=jax.ShapeDtypeStruct((n,), jnp.float32),
        mesh=mesh,
        scratch_types=[
            pltpu.VMEM((pw,), jnp.int32),
            pltpu.VMEM((2, _C, _D), jnp.float32),
            pltpu.VMEM((2, _C, 32), jnp.float32),
            pltpu.VMEM((_C, 256), jnp.float32),
            pltpu.VMEM((8, 32), jnp.float32),
            pltpu.VMEM((pw,), jnp.float32),
            pltpu.VMEM((16, 32), jnp.float32),
            pltpu.VMEM((16, 32), jnp.float32),
            pltpu.SemaphoreType.DMA,
            pltpu.SemaphoreType.DMA,
            pltpu.SemaphoreType.DMA,
        ],
        compiler_params=pltpu.CompilerParams(needs_layout_passes=False),
    )(body)
    return f(l, i12, r2all, mtab, r2_b)


def kernel(x_in, bb1_w, bb1_b, bb2_w, bb2_b, bb3_w, bb3_b,
           msk1_w, msk1_b, msk2_w, msk2_b, msk3_w, msk3_b,
           c1a_w, c1a_b, c1b_w, c1b_b, c1c_w, c1c_b,
           c2a_w, c2a_b, c2b_w, c2b_b, c3a_w, c3a_b, c3b_w, c3b_b,
           r1_w, r1_b, r2_w, r2_b, r3_w, r3_b):
    b, ch, hh, ww = x_in.shape
    xf = jnp.transpose(x_in, (0, 2, 3, 1)).reshape(_N, ch)
    c2aT = jnp.transpose(c2a_w, (1, 0, 2)).reshape(32, 512)
    c2bT = jnp.transpose(c2b_w, (1, 0, 2)).reshape(32, 512)
    r2T = jnp.transpose(r2_w, (1, 0, 2)).reshape(128, 256)
    msk3_wT = msk3_w.reshape(1, 16)
    def _pack_halves(x):
        lo = x[..., :16].astype(jnp.bfloat16)
        hi = x[..., 16:].astype(jnp.bfloat16)
        return jax.lax.bitcast_convert_type(
            jnp.stack([lo, hi], axis=-1), jnp.float32)

    j = jnp.arange(256, dtype=jnp.int32)
    rbase = jnp.clip(j * 16 - 8, 0, 4096 - 32)
    rows = rbase[:, None] + jnp.arange(32, dtype=jnp.int32)[None, :]
    r3flat = r3_w[:, :, 0]
    rblk = r3flat[rows]
    rbblk = r3_b[:, 0][rows]
    mtab = jnp.concatenate(
        [_pack_halves(c3a_w).reshape(256, 512), _pack_halves(c3a_b),
         _pack_halves(c3b_w).reshape(256, 512), _pack_halves(c3b_b),
         _pack_halves(rblk).reshape(256, 512), rbblk,
         jnp.zeros((256, 64), jnp.float32)], axis=1)

    nh = _N // 7
    outs = []
    masks = []
    for h in range(7):
        xf_h = xf[h * nh:(h + 1) * nh]
        mask_f, l, i12, r2all = _tc_stage(
            xf_h, bb1_w, bb1_b.reshape(1, 128), bb2_w, bb2_b.reshape(1, 128),
            bb3_w, bb3_b.reshape(1, 128),
            msk1_w, msk1_b.reshape(1, 32), msk2_w, msk2_b.reshape(1, 16),
            msk3_wT, msk3_b.reshape(1, 1),
            c1a_w, c1a_b.reshape(1, 32), c1b_w, c1b_b.reshape(1, 32),
            c1c_w, c1c_b.reshape(1, 16),
            c2aT, c2a_b, c2bT, c2b_b,
            r1_w, r1_b.reshape(1, 128), r2T)
        outs.append(_sc_stage(l, i12.reshape(nh), r2all, mtab, r2_b))
        masks.append(mask_f)
    out = jnp.concatenate(outs).reshape(b, 1, hh, ww)
    mask = jnp.concatenate(masks).reshape(b, 1, hh, ww)
    return out, mask
